# baseline (speedup 1.0000x reference)
import functools
import jax
import jax.numpy as jnp
import numpy as np
from jax import lax
from jax.experimental import pallas as pl
from jax.experimental.pallas import tpu as pltpu
from jax.experimental.pallas import tpu_sc as plsc

N = 10000
D = 128
H = 16
DH = 8
E = 160000
L = 4
R = 2
OUT = 64

BN = 400
GRID = N // BN
C = 32
NW = 32
KSTEPS = 159
NCHUNK = KSTEPS * NW
EP = NCHUNK * C
STRIPE = 640
FCH = 80
ND = 1280
NACC = N + 8
NER = 1256

_p = np.arange(128)
_S_PERM = ((_p % 16) * 8 + _p // 16).tolist()


def _dense_tail(h, Wt_ref, Wrt_ref, h1t_ref, feat_ref, er_ref):
    h1t_ref[...] = h
    for r in range(R):
        feat_ref[r] = jnp.dot(h, Wt_ref[r], preferred_element_type=jnp.float32)
        er_ref[r] = jnp.dot(h, Wrt_ref[r], preferred_element_type=jnp.float32)


def _embed_dense_body(x_ref, W1_ref, b1_ref, W2p_ref, b2p_ref, P_ref,
                      Wt_ref, Wrt_ref, h1t_ref, feat_ref, er_ref):
    x = x_ref[...]
    hmid = jnp.maximum(jnp.dot(x, W1_ref[...], preferred_element_type=jnp.float32)
                       + b1_ref[...], 0.0)
    h = (jnp.dot(hmid, W2p_ref[...], preferred_element_type=jnp.float32)
         + b2p_ref[...]
         + jnp.dot(x, P_ref[...], preferred_element_type=jnp.float32))
    _dense_tail(h, Wt_ref, Wrt_ref, h1t_ref, feat_ref, er_ref)


def _epilogue(outN_ref, outD_ref, h1t_ref):
    agg = jnp.zeros((BN, 128), jnp.float32)
    for r in range(R):
        num = outN_ref[r, 0] + outN_ref[r, 1]
        den = outD_ref[r, 0] + outD_ref[r, 1]
        dent = jnp.concatenate([den] * 8, axis=1) + 1e-9
        agg = agg + num / dent
    return jnp.where(agg >= 0, agg, 0.01 * agg) + h1t_ref[...]


def _epi_dense_body(outN_ref, outD_ref, h1t_ref, Wt_ref, Wrt_ref,
                    h1t_new_ref, feat_ref, er_ref):
    h = _epilogue(outN_ref, outD_ref, h1t_ref)
    _dense_tail(h, Wt_ref, Wrt_ref, h1t_new_ref, feat_ref, er_ref)


def _epi_decision_body(outN_ref, outD_ref, h1t_ref, Wd1p_ref, bd1_ref,
                       Wd2_ref, bd2_ref, out_ref):
    h = _epilogue(outN_ref, outD_ref, h1t_ref)
    hid = jnp.maximum(jnp.dot(h, Wd1p_ref[...], preferred_element_type=jnp.float32)
                      + bd1_ref[...], 0.0)
    out_ref[...] = jnp.dot(hid, Wd2_ref[...], preferred_element_type=jnp.float32) + bd2_ref[...]


_row_spec = pl.BlockSpec((BN, 128), lambda i: (i, 0))
_row16_spec = pl.BlockSpec((BN, 16), lambda i: (i, 0))
_w_spec = pl.BlockSpec((128, 128), lambda i: (0, 0))
_b_spec = pl.BlockSpec((1, 128), lambda i: (0, 0))
_Wt_spec = pl.BlockSpec((R, 128, 128), lambda i: (0, 0, 0))
_Wrt_spec = pl.BlockSpec((R, 128, 16), lambda i: (0, 0, 0))
_feat_spec = pl.BlockSpec((R, BN, 128), lambda i: (0, i, 0))
_er_spec = pl.BlockSpec((R, BN, 16), lambda i: (0, i, 0))
_accN_spec = pl.BlockSpec((R, 2, BN, 128), lambda i: (0, 0, i, 0))
_accD_spec = pl.BlockSpec((R, 2, BN, 16), lambda i: (0, 0, i, 0))

_dense_out_shapes = (
    jax.ShapeDtypeStruct((N, 128), jnp.float32),
    jax.ShapeDtypeStruct((R, N, 128), jnp.float32),
    jax.ShapeDtypeStruct((R, N, 16), jnp.float32),
)
_dense_out_specs = (_row_spec, _feat_spec, _er_spec)

_embed_dense = pl.pallas_call(
    _embed_dense_body,
    grid=(GRID,),
    in_specs=[_row_spec, _w_spec, _b_spec, _w_spec, _b_spec, _w_spec,
              _Wt_spec, _Wrt_spec],
    out_specs=_dense_out_specs,
    out_shape=_dense_out_shapes,
)

_epi_dense = pl.pallas_call(
    _epi_dense_body,
    grid=(GRID,),
    in_specs=[_accN_spec, _accD_spec, _row_spec, _Wt_spec, _Wrt_spec],
    out_specs=_dense_out_specs,
    out_shape=_dense_out_shapes,
)

_epi_decision = pl.pallas_call(
    _epi_decision_body,
    grid=(GRID,),
    in_specs=[_accN_spec, _accD_spec, _row_spec, _w_spec, _b_spec,
              pl.BlockSpec((128, OUT), lambda i: (0, 0)),
              pl.BlockSpec((1, OUT), lambda i: (0, 0))],
    out_specs=pl.BlockSpec((BN, OUT), lambda i: (i, 0)),
    out_shape=jax.ShapeDtypeStruct((N, OUT), jnp.float32),
)


def _sc_body(tab_hbm, alt_hbm, gi0_hbm, dst0_hbm, gi1_hbm, dst1_hbm,
             zeros_hbm, outN_hbm, outD_hbm,
             gbuf, exrow, gidxb, dstb, d2s, alv,
             gsem, ssemN, ssemD, isem,
             accN, accD):
    idx_hbms = ((gi0_hbm, dst0_hbm), (gi1_hbm, dst1_hbm))
    cid = lax.axis_index("c")
    sid = lax.axis_index("s")
    wid = sid * 2 + cid
    zero16 = jnp.zeros((16,), jnp.float32)

    pltpu.sync_copy(alt_hbm, alv)

    def _zf(i, _):
        for sl in range(3):
            for j in range(8):
                exrow[sl, i, pl.ds(j * 16, 16)] = zero16
        return _
    lax.fori_loop(0, C, _zf, None)

    for r in range(R):
        gi_hbm, dst_hbm = idx_hbms[r]

        @pl.when(sid < 15)
        def _():
            pltpu.sync_copy(zeros_hbm, accN.at[pl.ds(sid * STRIPE, STRIPE)])

        @pl.when(sid == 15)
        def _():
            pltpu.sync_copy(zeros_hbm.at[pl.ds(0, NACC - 15 * STRIPE)],
                            accN.at[pl.ds(15 * STRIPE, NACC - 15 * STRIPE)])
        pltpu.sync_copy(zeros_hbm.at[pl.ds(0, FCH)], accD.at[pl.ds(sid * FCH, FCH)])
        plsc.subcore_barrier()

        def _idx_load(k):
            s = k % 8
            ck = wid + NW * k
            pltpu.async_copy(gi_hbm.at[pl.ds(ck * 2 * C, 2 * C)], gidxb.at[s],
                             isem.at[s])
            pltpu.async_copy(dst_hbm.at[pl.ds(ck * C, C)], dstb.at[s], isem.at[s])

        def _gather(k, slot):
            s = k % 8
            ck = wid + NW * k
            pltpu.make_async_copy(gi_hbm.at[pl.ds(ck * 2 * C, 2 * C)],
                                  gidxb.at[s], isem.at[s]).wait()
            pltpu.make_async_copy(dst_hbm.at[pl.ds(ck * C, C)], dstb.at[s],
                                  isem.at[s]).wait()
            for k2 in range(C // 16):
                d2s[s, pl.ds(k2 * 16, 16)] = lax.shift_right_logical(
                    dstb[s, pl.ds(k2 * 16, 16)], 3)
            pltpu.async_copy(tab_hbm.at[r].at[gidxb.at[s]], gbuf.at[slot],
                             gsem.at[slot])

        def _gather_wait(k, slot):
            s = k % 8
            pltpu.make_async_copy(tab_hbm.at[r].at[gidxb.at[s]], gbuf.at[slot],
                                  gsem.at[slot]).wait()

        def _scatter(k, slot):
            s = k % 8
            pltpu.async_copy(gbuf.at[slot, pl.ds(0, C)], accN.at[dstb.at[s]],
                             ssemN.at[slot], add=True)
            pltpu.async_copy(exrow.at[slot], accD.at[d2s.at[s]], ssemD.at[slot],
                             add=True)

        def _retire(k, slot):
            s = k % 8
            pltpu.make_async_copy(gbuf.at[slot, pl.ds(0, C)], accN.at[dstb.at[s]],
                                  ssemN.at[slot]).wait()
            pltpu.make_async_copy(exrow.at[slot], accD.at[d2s.at[s]],
                                  ssemD.at[slot]).wait()

            def _zb(k2, _):
                dvz = (dstb[s, pl.ds(k2 * 16, 16)] & 7) * 16
                for m in range(16):
                    exrow[slot, k2 * 16 + m, pl.ds(dvz[m], 16)] = zero16
                return _
            lax.fori_loop(0, C // 16, _zb, None)

        _idx_load(0)
        _idx_load(1)
        _idx_load(2)
        _gather(0, 0)

        def _step(k, _):
            b = k % 3
            pb = (k + 1) % 3
            s = k % 8
            _gather_wait(k, b)

            def _e16(k2, _):
                dv = (dstb[s, pl.ds(k2 * 16, 16)] & 7) * 16
                for m in range(16):
                    i = k2 * 16 + m
                    off = dv[m]
                    fs = [gbuf[b, i, pl.ds(j * 16, 16)] for j in range(8)]
                    el = fs[0] * alv[pl.ds(r * 128, 16)]
                    for j in range(1, 8):
                        el = el + fs[j] * alv[pl.ds(r * 128 + j * 16, 16)]
                    e = el + gbuf[b, C + i, pl.ds(off, 16)]
                    e = jnp.where(e >= 0.0, e, 0.2 * e)
                    ex = jnp.exp(e)
                    for j in range(8):
                        gbuf[b, i, pl.ds(j * 16, 16)] = fs[j] * ex
                    exrow[b, i, pl.ds(off, 16)] = ex
                return _
            lax.fori_loop(0, C // 16, _e16, None)

            _scatter(k, b)

            @pl.when(k >= 2)
            def _():
                _retire(k - 2, pb)

            @pl.when(k + 1 <= KSTEPS - 1)
            def _():
                _gather(k + 1, pb)

            @pl.when(k + 3 <= KSTEPS - 1)
            def _():
                _idx_load(k + 3)
            return _

        lax.fori_loop(0, KSTEPS, _step, None)

        _retire(KSTEPS - 2, (KSTEPS - 2) % 3)
        _retire(KSTEPS - 1, (KSTEPS - 1) % 3)
        plsc.subcore_barrier()

        for cpy in range(STRIPE // FCH):
            start = sid * STRIPE + cpy * FCH

            @pl.when(start < N)
            def _():
                pltpu.sync_copy(accN.at[pl.ds(start, FCH)],
                                outN_hbm.at[r, cid, pl.ds(start, FCH)])
        pltpu.sync_copy(accD.at[pl.ds(sid * FCH, FCH)],
                        outD_hbm.at[r, cid, pl.ds(sid * FCH, FCH)])
        plsc.subcore_barrier()


_sc_edge = pl.kernel(
    _sc_body,
    out_type=(
        jax.ShapeDtypeStruct((R, 2, N, 128), jnp.float32),
        jax.ShapeDtypeStruct((R, 2, ND, 128), jnp.float32),
    ),
    mesh=plsc.VectorSubcoreMesh(core_axis_name="c", subcore_axis_name="s",
                                num_cores=2, num_subcores=16),
    scratch_types=[
        pltpu.VMEM((3, 2 * C, 128), jnp.float32),
        pltpu.VMEM((3, C, 128), jnp.float32),
        pltpu.VMEM((8, 2 * C), jnp.int32),
        pltpu.VMEM((8, C), jnp.int32),
        pltpu.VMEM((8, C), jnp.int32),
        pltpu.VMEM((R * 128,), jnp.float32),
        pltpu.SemaphoreType.DMA((3,)),
        pltpu.SemaphoreType.DMA((3,)),
        pltpu.SemaphoreType.DMA((3,)),
        pltpu.SemaphoreType.DMA((8,)),
        pltpu.VMEM_SHARED((NACC, 128), jnp.float32),
        pltpu.VMEM_SHARED((ND, 128), jnp.float32),
    ],
)


@jax.jit
def kernel(inputs, edge_index_rel0, edge_index_rel1, W_emb1, b_emb1, W_emb2,
           b_emb2, W_gat, a_l, a_r, W_dec1, b_dec1, W_dec2, b_dec2):
    sp = jnp.asarray(_S_PERM)
    P = jnp.eye(128, dtype=jnp.float32)[sp].T
    W2p = W_emb2[:, sp]
    b2p = b_emb2[sp].reshape(1, 128)
    Wt = W_gat[:, :, sp][:, :, :, sp]
    Wr_ = jnp.einsum('lrkhd,lrhd->lrkh', W_gat.reshape(L, R, 128, H, DH), a_r)
    Wrt = Wr_[:, :, sp, :]
    alt = a_l.transpose(0, 1, 3, 2).reshape(L, R, 128)
    Wd1p = W_dec1[sp]

    def _prep(ei):
        srcp = jnp.concatenate([ei[0], jnp.zeros((EP - E,), jnp.int32)])
        dstp = jnp.concatenate([ei[1], jnp.full((EP - E,), N, jnp.int32)])
        gi = jnp.concatenate([srcp.reshape(NCHUNK, C),
                              NACC + (dstp >> 3).reshape(NCHUNK, C)],
                             axis=1).reshape(NCHUNK * 2 * C)
        return gi, dstp

    gi0, dst0 = _prep(edge_index_rel0)
    gi1, dst1 = _prep(edge_index_rel1)
    zeros = jnp.zeros((STRIPE, 128), jnp.float32)

    h1t, feat, er = _embed_dense(inputs, W_emb1, b_emb1.reshape(1, 128),
                                 W2p, b2p, P, Wt[0], Wrt[0])
    for l in range(L):
        er_pack = jnp.pad(er.reshape(R, N // 8, 128),
                          ((0, 0), (0, NER - N // 8), (0, 0)))
        tab = jnp.concatenate(
            [feat, jnp.zeros((R, NACC - N, 128), jnp.float32), er_pack], axis=1)
        outN, outDp = _sc_edge(tab, alt[l].reshape(R * 128),
                               gi0, dst0, gi1, dst1, zeros)
        outD = outDp.reshape(R, 2, ND * 8, 16)[:, :, :N]
        if l + 1 < L:
            h1t, feat, er = _epi_dense(outN, outD, h1t, Wt[l + 1], Wrt[l + 1])
    return _epi_decision(outN, outD, h1t, Wd1p, b_dec1.reshape(1, 128),
                         W_dec2, b_dec2.reshape(1, OUT))


# speedup vs baseline: 1.0585x; 1.0585x over previous
import functools
import jax
import jax.numpy as jnp
import numpy as np
from jax import lax
from jax.experimental import pallas as pl
from jax.experimental.pallas import tpu as pltpu
from jax.experimental.pallas import tpu_sc as plsc

N = 10000
D = 128
H = 16
DH = 8
E = 160000
L = 4
R = 2
OUT = 64

BN = 400
GRID = N // BN
C = 32
NW = 32
KSTEPS = 159
NCHUNK = KSTEPS * NW
EP = NCHUNK * C
STRIPE = 640
FCH = 80
ND = 1280
NACC = N + 8
NER = 1256

_p = np.arange(128)
_S_PERM = ((_p % 16) * 8 + _p // 16).tolist()


def _dense_tail(h, Wt_ref, Wrt_ref, h1t_ref, feat_ref, er_ref):
    h1t_ref[...] = h
    for r in range(R):
        feat_ref[r] = jnp.dot(h, Wt_ref[r], preferred_element_type=jnp.float32)
        er_ref[r] = jnp.dot(h, Wrt_ref[r], preferred_element_type=jnp.float32)


def _embed_dense_body(x_ref, W1_ref, b1_ref, W2p_ref, b2p_ref, P_ref,
                      Wt_ref, Wrt_ref, h1t_ref, feat_ref, er_ref):
    x = x_ref[...]
    hmid = jnp.maximum(jnp.dot(x, W1_ref[...], preferred_element_type=jnp.float32)
                       + b1_ref[...], 0.0)
    h = (jnp.dot(hmid, W2p_ref[...], preferred_element_type=jnp.float32)
         + b2p_ref[...]
         + jnp.dot(x, P_ref[...], preferred_element_type=jnp.float32))
    _dense_tail(h, Wt_ref, Wrt_ref, h1t_ref, feat_ref, er_ref)


def _epilogue(outN_ref, outD_ref, h1t_ref):
    agg = jnp.zeros((BN, 128), jnp.float32)
    for r in range(R):
        num = outN_ref[r, 0] + outN_ref[r, 1]
        den = outD_ref[r, 0] + outD_ref[r, 1]
        dent = jnp.concatenate([den] * 8, axis=1) + 1e-9
        agg = agg + num / dent
    return jnp.where(agg >= 0, agg, 0.01 * agg) + h1t_ref[...]


def _epi_dense_body(outN_ref, outD_ref, h1t_ref, Wt_ref, Wrt_ref,
                    h1t_new_ref, feat_ref, er_ref):
    h = _epilogue(outN_ref, outD_ref, h1t_ref)
    _dense_tail(h, Wt_ref, Wrt_ref, h1t_new_ref, feat_ref, er_ref)


def _epi_decision_body(outN_ref, outD_ref, h1t_ref, Wd1p_ref, bd1_ref,
                       Wd2_ref, bd2_ref, out_ref):
    h = _epilogue(outN_ref, outD_ref, h1t_ref)
    hid = jnp.maximum(jnp.dot(h, Wd1p_ref[...], preferred_element_type=jnp.float32)
                      + bd1_ref[...], 0.0)
    out_ref[...] = jnp.dot(hid, Wd2_ref[...], preferred_element_type=jnp.float32) + bd2_ref[...]


_row_spec = pl.BlockSpec((BN, 128), lambda i: (i, 0))
_row16_spec = pl.BlockSpec((BN, 16), lambda i: (i, 0))
_w_spec = pl.BlockSpec((128, 128), lambda i: (0, 0))
_b_spec = pl.BlockSpec((1, 128), lambda i: (0, 0))
_Wt_spec = pl.BlockSpec((R, 128, 128), lambda i: (0, 0, 0))
_Wrt_spec = pl.BlockSpec((R, 128, 16), lambda i: (0, 0, 0))
_feat_spec = pl.BlockSpec((R, BN, 128), lambda i: (0, i, 0))
_er_spec = pl.BlockSpec((R, BN, 16), lambda i: (0, i, 0))
_accN_spec = pl.BlockSpec((R, 2, BN, 128), lambda i: (0, 0, i, 0))
_accD_spec = pl.BlockSpec((R, 2, BN, 16), lambda i: (0, 0, i, 0))

_dense_out_shapes = (
    jax.ShapeDtypeStruct((N, 128), jnp.float32),
    jax.ShapeDtypeStruct((R, N, 128), jnp.float32),
    jax.ShapeDtypeStruct((R, N, 16), jnp.float32),
)
_dense_out_specs = (_row_spec, _feat_spec, _er_spec)

_embed_dense = pl.pallas_call(
    _embed_dense_body,
    grid=(GRID,),
    in_specs=[_row_spec, _w_spec, _b_spec, _w_spec, _b_spec, _w_spec,
              _Wt_spec, _Wrt_spec],
    out_specs=_dense_out_specs,
    out_shape=_dense_out_shapes,
)

_epi_dense = pl.pallas_call(
    _epi_dense_body,
    grid=(GRID,),
    in_specs=[_accN_spec, _accD_spec, _row_spec, _Wt_spec, _Wrt_spec],
    out_specs=_dense_out_specs,
    out_shape=_dense_out_shapes,
)

_epi_decision = pl.pallas_call(
    _epi_decision_body,
    grid=(GRID,),
    in_specs=[_accN_spec, _accD_spec, _row_spec, _w_spec, _b_spec,
              pl.BlockSpec((128, OUT), lambda i: (0, 0)),
              pl.BlockSpec((1, OUT), lambda i: (0, 0))],
    out_specs=pl.BlockSpec((BN, OUT), lambda i: (i, 0)),
    out_shape=jax.ShapeDtypeStruct((N, OUT), jnp.float32),
)


def _sc_body(feat_hbm, er_hbm, alt_hbm, src0_hbm, dst0_hbm, d20_hbm,
             src1_hbm, dst1_hbm, d21_hbm, zeros_hbm,
             outN_hbm, outD_hbm,
             featbuf, erbuf, exrow, srcb, dstb, d2b, alv,
             gsemF, gsemE, ssemN, ssemD, isem,
             accN, accD):
    idx_hbms = ((src0_hbm, dst0_hbm, d20_hbm), (src1_hbm, dst1_hbm, d21_hbm))
    cid = lax.axis_index("c")
    sid = lax.axis_index("s")
    wid = sid * 2 + cid
    zero16 = jnp.zeros((16,), jnp.float32)

    pltpu.sync_copy(alt_hbm, alv)

    def _zf(i, _):
        for sl in range(3):
            for j in range(8):
                exrow[sl, i, pl.ds(j * 16, 16)] = zero16
        return _
    lax.fori_loop(0, C, _zf, None)

    for r in range(R):
        src_hbm, dst_hbm, d2_hbm = idx_hbms[r]

        @pl.when(sid < 15)
        def _():
            pltpu.sync_copy(zeros_hbm, accN.at[pl.ds(sid * STRIPE, STRIPE)])

        @pl.when(sid == 15)
        def _():
            pltpu.sync_copy(zeros_hbm.at[pl.ds(0, NACC - 15 * STRIPE)],
                            accN.at[pl.ds(15 * STRIPE, NACC - 15 * STRIPE)])
        pltpu.sync_copy(zeros_hbm.at[pl.ds(0, FCH)], accD.at[pl.ds(sid * FCH, FCH)])
        plsc.subcore_barrier()

        def _idx_load(k):
            s = k % 8
            base = (wid + NW * k) * C
            pltpu.async_copy(src_hbm.at[pl.ds(base, C)], srcb.at[s], isem.at[s])
            pltpu.async_copy(dst_hbm.at[pl.ds(base, C)], dstb.at[s], isem.at[s])
            pltpu.async_copy(d2_hbm.at[pl.ds(base, C)], d2b.at[s], isem.at[s])

        def _idx_wait(k):
            s = k % 8
            base = (wid + NW * k) * C
            pltpu.make_async_copy(src_hbm.at[pl.ds(base, C)], srcb.at[s], isem.at[s]).wait()
            pltpu.make_async_copy(dst_hbm.at[pl.ds(base, C)], dstb.at[s], isem.at[s]).wait()
            pltpu.make_async_copy(d2_hbm.at[pl.ds(base, C)], d2b.at[s], isem.at[s]).wait()

        def _gather(k, slot):
            s = k % 8
            _idx_wait(k)
            pltpu.async_copy(feat_hbm.at[r].at[srcb.at[s]], featbuf.at[slot],
                             gsemF.at[slot])
            pltpu.async_copy(er_hbm.at[r].at[d2b.at[s]], erbuf.at[slot],
                             gsemE.at[slot])

        def _gather_wait(k, slot):
            s = k % 8
            pltpu.make_async_copy(feat_hbm.at[r].at[srcb.at[s]], featbuf.at[slot],
                                  gsemF.at[slot]).wait()
            pltpu.make_async_copy(er_hbm.at[r].at[d2b.at[s]], erbuf.at[slot],
                                  gsemE.at[slot]).wait()

        def _scatter(k, slot):
            s = k % 8
            pltpu.async_copy(featbuf.at[slot], accN.at[dstb.at[s]], ssemN.at[slot],
                             add=True)
            pltpu.async_copy(exrow.at[slot], accD.at[d2b.at[s]], ssemD.at[slot],
                             add=True)

        def _retire(k, slot):
            s = k % 8
            pltpu.make_async_copy(featbuf.at[slot], accN.at[dstb.at[s]],
                                  ssemN.at[slot]).wait()
            pltpu.make_async_copy(exrow.at[slot], accD.at[d2b.at[s]],
                                  ssemD.at[slot]).wait()

            def _zb(k2, _):
                dvz = (dstb[s, pl.ds(k2 * 16, 16)] & 7) * 16
                for m in range(16):
                    exrow[slot, k2 * 16 + m, pl.ds(dvz[m], 16)] = zero16
                return _
            lax.fori_loop(0, C // 16, _zb, None)

        _idx_load(0)
        _idx_load(1)
        _idx_load(2)
        _gather(0, 0)

        al_vecs = [alv[pl.ds(r * 128 + j * 16, 16)] for j in range(8)]

        def _step(k, _):
            b = k % 3
            pb = (k + 1) % 3
            s = k % 8
            _gather_wait(k, b)

            def _e16(k2, _):
                dv = (dstb[s, pl.ds(k2 * 16, 16)] & 7) * 16
                for m in range(16):
                    i = k2 * 16 + m
                    off = dv[m]
                    fs = [featbuf[b, i, pl.ds(j * 16, 16)] for j in range(8)]
                    el = fs[0] * al_vecs[0]
                    for j in range(1, 8):
                        el = el + fs[j] * al_vecs[j]
                    e = el + erbuf[b, i, pl.ds(off, 16)]
                    e = jnp.where(e >= 0.0, e, 0.2 * e)
                    ex = jnp.exp(e)
                    for j in range(8):
                        featbuf[b, i, pl.ds(j * 16, 16)] = fs[j] * ex
                    exrow[b, i, pl.ds(off, 16)] = ex
                return _
            lax.fori_loop(0, C // 16, _e16, None, unroll=2)

            _scatter(k, b)

            @pl.when(k >= 2)
            def _():
                _retire(k - 2, pb)

            @pl.when(k + 1 <= KSTEPS - 1)
            def _():
                _gather(k + 1, pb)

            @pl.when(k + 3 <= KSTEPS - 1)
            def _():
                _idx_load(k + 3)
            return _

        lax.fori_loop(0, KSTEPS, _step, None)

        _retire(KSTEPS - 2, (KSTEPS - 2) % 3)
        _retire(KSTEPS - 1, (KSTEPS - 1) % 3)
        plsc.subcore_barrier()

        for cpy in range(STRIPE // FCH):
            start = sid * STRIPE + cpy * FCH

            @pl.when(start < N)
            def _():
                pltpu.sync_copy(accN.at[pl.ds(start, FCH)],
                                outN_hbm.at[r, cid, pl.ds(start, FCH)])
        pltpu.sync_copy(accD.at[pl.ds(sid * FCH, FCH)],
                        outD_hbm.at[r, cid, pl.ds(sid * FCH, FCH)])
        plsc.subcore_barrier()


_sc_edge = pl.kernel(
    _sc_body,
    out_type=(
        jax.ShapeDtypeStruct((R, 2, N, 128), jnp.float32),
        jax.ShapeDtypeStruct((R, 2, ND, 128), jnp.float32),
    ),
    mesh=plsc.VectorSubcoreMesh(core_axis_name="c", subcore_axis_name="s",
                                num_cores=2, num_subcores=16),
    scratch_types=[
        pltpu.VMEM((3, C, 128), jnp.float32),
        pltpu.VMEM((3, C, 128), jnp.float32),
        pltpu.VMEM((3, C, 128), jnp.float32),
        pltpu.VMEM((8, C), jnp.int32),
        pltpu.VMEM((8, C), jnp.int32),
        pltpu.VMEM((8, C), jnp.int32),
        pltpu.VMEM((R * 128,), jnp.float32),
        pltpu.SemaphoreType.DMA((3,)),
        pltpu.SemaphoreType.DMA((3,)),
        pltpu.SemaphoreType.DMA((3,)),
        pltpu.SemaphoreType.DMA((3,)),
        pltpu.SemaphoreType.DMA((8,)),
        pltpu.VMEM_SHARED((NACC, 128), jnp.float32),
        pltpu.VMEM_SHARED((ND, 128), jnp.float32),
    ],
)


@jax.jit
def kernel(inputs, edge_index_rel0, edge_index_rel1, W_emb1, b_emb1, W_emb2,
           b_emb2, W_gat, a_l, a_r, W_dec1, b_dec1, W_dec2, b_dec2):
    sp = jnp.asarray(_S_PERM)
    P = jnp.eye(128, dtype=jnp.float32)[sp].T
    W2p = W_emb2[:, sp]
    b2p = b_emb2[sp].reshape(1, 128)
    Wt = W_gat[:, :, sp][:, :, :, sp]
    Wr_ = jnp.einsum('lrkhd,lrhd->lrkh', W_gat.reshape(L, R, 128, H, DH), a_r)
    Wrt = Wr_[:, :, sp, :]
    alt = a_l.transpose(0, 1, 3, 2).reshape(L, R, 128)
    Wd1p = W_dec1[sp]

    def _prep(ei):
        srcp = jnp.concatenate([ei[0], jnp.zeros((EP - E,), jnp.int32)])
        dstp = jnp.concatenate([ei[1], jnp.full((EP - E,), N, jnp.int32)])
        return srcp, dstp, dstp >> 3

    src0, dst0, d20 = _prep(edge_index_rel0)
    src1, dst1, d21 = _prep(edge_index_rel1)
    zeros = jnp.zeros((STRIPE, 128), jnp.float32)

    h1t, feat, er = _embed_dense(inputs, W_emb1, b_emb1.reshape(1, 128),
                                 W2p, b2p, P, Wt[0], Wrt[0])
    for l in range(L):
        er_pack = jnp.pad(er.reshape(R, N * 16 // 128, 128),
                          ((0, 0), (0, NER - N * 16 // 128), (0, 0)))
        outN, outDp = _sc_edge(feat, er_pack, alt[l].reshape(R * 128),
                               src0, dst0, d20, src1, dst1, d21, zeros)
        outD = outDp.reshape(R, 2, ND * 8, 16)[:, :, :N]
        if l + 1 < L:
            h1t, feat, er = _epi_dense(outN, outD, h1t, Wt[l + 1], Wrt[l + 1])
    return _epi_decision(outN, outD, h1t, Wd1p, b_dec1.reshape(1, 128),
                         W_dec2, b_dec2.reshape(1, OUT))


# speedup vs baseline: 1.0897x; 1.0294x over previous
import functools
import jax
import jax.numpy as jnp
import numpy as np
from jax import lax
from jax.experimental import pallas as pl
from jax.experimental.pallas import tpu as pltpu
from jax.experimental.pallas import tpu_sc as plsc

N = 10000
D = 128
H = 16
DH = 8
E = 160000
L = 4
R = 2
OUT = 64

BN = 400
GRID = N // BN
C = 32
NW = 32
KSTEPS = 159
NCHUNK = KSTEPS * NW
EP = NCHUNK * C
STRIPE = 640
FCH = 80
ND = 1280
NACC = N + 8
NER = 1256

_p = np.arange(128)
_S_PERM = ((_p % 16) * 8 + _p // 16).tolist()


def _dense_tail(h, Wt_ref, Wrt_ref, h1t_ref, feat_ref, er_ref):
    h1t_ref[...] = h
    for r in range(R):
        feat_ref[r] = jnp.dot(h, Wt_ref[r], preferred_element_type=jnp.float32)
        er_ref[r] = jnp.dot(h, Wrt_ref[r], preferred_element_type=jnp.float32)


def _embed_dense_body(x_ref, W1_ref, b1_ref, W2p_ref, b2p_ref, P_ref,
                      Wt_ref, Wrt_ref, h1t_ref, feat_ref, er_ref):
    x = x_ref[...]
    hmid = jnp.maximum(jnp.dot(x, W1_ref[...], preferred_element_type=jnp.float32)
                       + b1_ref[...], 0.0)
    h = (jnp.dot(hmid, W2p_ref[...], preferred_element_type=jnp.float32)
         + b2p_ref[...]
         + jnp.dot(x, P_ref[...], preferred_element_type=jnp.float32))
    _dense_tail(h, Wt_ref, Wrt_ref, h1t_ref, feat_ref, er_ref)


def _epilogue(outN_ref, outD_ref, h1t_ref):
    agg = jnp.zeros((BN, 128), jnp.float32)
    for r in range(R):
        num = outN_ref[r, 0] + outN_ref[r, 1]
        den = outD_ref[r, 0] + outD_ref[r, 1]
        dent = jnp.concatenate([den] * 8, axis=1) + 1e-9
        agg = agg + num / dent
    return jnp.where(agg >= 0, agg, 0.01 * agg) + h1t_ref[...]


def _epi_dense_body(outN_ref, outD_ref, h1t_ref, Wt_ref, Wrt_ref,
                    h1t_new_ref, feat_ref, er_ref):
    h = _epilogue(outN_ref, outD_ref, h1t_ref)
    _dense_tail(h, Wt_ref, Wrt_ref, h1t_new_ref, feat_ref, er_ref)


def _epi_decision_body(outN_ref, outD_ref, h1t_ref, Wd1p_ref, bd1_ref,
                       Wd2_ref, bd2_ref, out_ref):
    h = _epilogue(outN_ref, outD_ref, h1t_ref)
    hid = jnp.maximum(jnp.dot(h, Wd1p_ref[...], preferred_element_type=jnp.float32)
                      + bd1_ref[...], 0.0)
    out_ref[...] = jnp.dot(hid, Wd2_ref[...], preferred_element_type=jnp.float32) + bd2_ref[...]


_row_spec = pl.BlockSpec((BN, 128), lambda i: (i, 0))
_row16_spec = pl.BlockSpec((BN, 16), lambda i: (i, 0))
_w_spec = pl.BlockSpec((128, 128), lambda i: (0, 0))
_b_spec = pl.BlockSpec((1, 128), lambda i: (0, 0))
_Wt_spec = pl.BlockSpec((R, 128, 128), lambda i: (0, 0, 0))
_Wrt_spec = pl.BlockSpec((R, 128, 16), lambda i: (0, 0, 0))
_feat_spec = pl.BlockSpec((R, BN, 128), lambda i: (0, i, 0))
_er_spec = pl.BlockSpec((R, BN, 16), lambda i: (0, i, 0))
_accN_spec = pl.BlockSpec((R, 2, BN, 128), lambda i: (0, 0, i, 0))
_accD_spec = pl.BlockSpec((R, 2, BN, 16), lambda i: (0, 0, i, 0))

_dense_out_shapes = (
    jax.ShapeDtypeStruct((N, 128), jnp.float32),
    jax.ShapeDtypeStruct((R, N, 128), jnp.float32),
    jax.ShapeDtypeStruct((R, N, 16), jnp.float32),
)
_dense_out_specs = (_row_spec, _feat_spec, _er_spec)

_embed_dense = pl.pallas_call(
    _embed_dense_body,
    grid=(GRID,),
    in_specs=[_row_spec, _w_spec, _b_spec, _w_spec, _b_spec, _w_spec,
              _Wt_spec, _Wrt_spec],
    out_specs=_dense_out_specs,
    out_shape=_dense_out_shapes,
)

_epi_dense = pl.pallas_call(
    _epi_dense_body,
    grid=(GRID,),
    in_specs=[_accN_spec, _accD_spec, _row_spec, _Wt_spec, _Wrt_spec],
    out_specs=_dense_out_specs,
    out_shape=_dense_out_shapes,
)

_epi_decision = pl.pallas_call(
    _epi_decision_body,
    grid=(GRID,),
    in_specs=[_accN_spec, _accD_spec, _row_spec, _w_spec, _b_spec,
              pl.BlockSpec((128, OUT), lambda i: (0, 0)),
              pl.BlockSpec((1, OUT), lambda i: (0, 0))],
    out_specs=pl.BlockSpec((BN, OUT), lambda i: (i, 0)),
    out_shape=jax.ShapeDtypeStruct((N, OUT), jnp.float32),
)


def _sc_body(feat_hbm, er_hbm, alt_hbm, src0_hbm, dst0_hbm, d20_hbm,
             src1_hbm, dst1_hbm, d21_hbm, zeros_hbm,
             outN_hbm, outD_hbm,
             featbuf, erbuf, exrow, srcb, dstb, d2b, alv,
             gsemF, gsemE, ssemN, ssemD, isem,
             accN, accD):
    idx_hbms = ((src0_hbm, dst0_hbm, d20_hbm), (src1_hbm, dst1_hbm, d21_hbm))
    cid = lax.axis_index("c")
    sid = lax.axis_index("s")
    wid = sid * 2 + cid
    zero16 = jnp.zeros((16,), jnp.float32)

    pltpu.sync_copy(alt_hbm, alv)

    def _zf(i, _):
        for sl in range(3):
            for j in range(8):
                exrow[sl, i, pl.ds(j * 16, 16)] = zero16
        return _
    lax.fori_loop(0, C, _zf, None)

    for r in range(R):
        src_hbm, dst_hbm, d2_hbm = idx_hbms[r]

        @pl.when(sid < 15)
        def _():
            pltpu.sync_copy(zeros_hbm, accN.at[pl.ds(sid * STRIPE, STRIPE)])

        @pl.when(sid == 15)
        def _():
            pltpu.sync_copy(zeros_hbm.at[pl.ds(0, NACC - 15 * STRIPE)],
                            accN.at[pl.ds(15 * STRIPE, NACC - 15 * STRIPE)])
        pltpu.sync_copy(zeros_hbm.at[pl.ds(0, FCH)], accD.at[pl.ds(sid * FCH, FCH)])
        plsc.subcore_barrier()

        def _idx_load(k):
            s = k % 8
            base = (wid + NW * k) * C
            pltpu.async_copy(src_hbm.at[pl.ds(base, C)], srcb.at[s], isem.at[s])
            pltpu.async_copy(dst_hbm.at[pl.ds(base, C)], dstb.at[s], isem.at[s])
            pltpu.async_copy(d2_hbm.at[pl.ds(base, C)], d2b.at[s], isem.at[s])

        def _idx_wait(k):
            s = k % 8
            base = (wid + NW * k) * C
            pltpu.make_async_copy(src_hbm.at[pl.ds(base, C)], srcb.at[s], isem.at[s]).wait()
            pltpu.make_async_copy(dst_hbm.at[pl.ds(base, C)], dstb.at[s], isem.at[s]).wait()
            pltpu.make_async_copy(d2_hbm.at[pl.ds(base, C)], d2b.at[s], isem.at[s]).wait()

        def _gather(k, slot):
            s = k % 8
            _idx_wait(k)
            pltpu.async_copy(feat_hbm.at[r].at[srcb.at[s]], featbuf.at[slot],
                             gsemF.at[slot])
            pltpu.async_copy(er_hbm.at[r].at[dstb.at[s]], erbuf.at[slot],
                             gsemE.at[slot])

        def _gather_wait(k, slot):
            s = k % 8
            pltpu.make_async_copy(feat_hbm.at[r].at[srcb.at[s]], featbuf.at[slot],
                                  gsemF.at[slot]).wait()
            pltpu.make_async_copy(er_hbm.at[r].at[dstb.at[s]], erbuf.at[slot],
                                  gsemE.at[slot]).wait()

        def _scatter(k, slot):
            s = k % 8
            pltpu.async_copy(featbuf.at[slot], accN.at[dstb.at[s]], ssemN.at[slot],
                             add=True)
            pltpu.async_copy(exrow.at[slot], accD.at[d2b.at[s]], ssemD.at[slot],
                             add=True)

        def _retire(k, slot):
            s = k % 8
            pltpu.make_async_copy(featbuf.at[slot], accN.at[dstb.at[s]],
                                  ssemN.at[slot]).wait()
            pltpu.make_async_copy(exrow.at[slot], accD.at[d2b.at[s]],
                                  ssemD.at[slot]).wait()

            def _zb(k2, _):
                dvz = (dstb[s, pl.ds(k2 * 16, 16)] & 7) * 16
                for m in range(16):
                    exrow[slot, k2 * 16 + m, pl.ds(dvz[m], 16)] = zero16
                return _
            lax.fori_loop(0, C // 16, _zb, None)

        _idx_load(0)
        _idx_load(1)
        _idx_load(2)
        _gather(0, 0)

        al_vecs = [alv[pl.ds(r * 128 + j * 16, 16)] for j in range(8)]

        def _step(k, _):
            b = k % 3
            pb = (k + 1) % 3
            s = k % 8
            _gather_wait(k, b)

            def _e16(k2, _):
                dv = (dstb[s, pl.ds(k2 * 16, 16)] & 7) * 16
                for m in range(16):
                    i = k2 * 16 + m
                    off = dv[m]
                    fs = [featbuf[b, i, pl.ds(j * 16, 16)] for j in range(8)]
                    el = fs[0] * al_vecs[0]
                    for j in range(1, 8):
                        el = el + fs[j] * al_vecs[j]
                    e = el + erbuf[b, i]
                    e = jnp.where(e >= 0.0, e, 0.2 * e)
                    ex = jnp.exp(e)
                    for j in range(8):
                        featbuf[b, i, pl.ds(j * 16, 16)] = fs[j] * ex
                    exrow[b, i, pl.ds(off, 16)] = ex
                return _
            lax.fori_loop(0, C // 16, _e16, None, unroll=2)

            _scatter(k, b)

            @pl.when(k >= 2)
            def _():
                _retire(k - 2, pb)

            @pl.when(k + 1 <= KSTEPS - 1)
            def _():
                _gather(k + 1, pb)

            @pl.when(k + 3 <= KSTEPS - 1)
            def _():
                _idx_load(k + 3)
            return _

        lax.fori_loop(0, KSTEPS, _step, None)

        _retire(KSTEPS - 2, (KSTEPS - 2) % 3)
        _retire(KSTEPS - 1, (KSTEPS - 1) % 3)
        plsc.subcore_barrier()

        for cpy in range(STRIPE // FCH):
            start = sid * STRIPE + cpy * FCH

            @pl.when(start < N)
            def _():
                pltpu.sync_copy(accN.at[pl.ds(start, FCH)],
                                outN_hbm.at[r, cid, pl.ds(start, FCH)])
        pltpu.sync_copy(accD.at[pl.ds(sid * FCH, FCH)],
                        outD_hbm.at[r, cid, pl.ds(sid * FCH, FCH)])
        plsc.subcore_barrier()


_sc_edge = pl.kernel(
    _sc_body,
    out_type=(
        jax.ShapeDtypeStruct((R, 2, N, 128), jnp.float32),
        jax.ShapeDtypeStruct((R, 2, ND, 128), jnp.float32),
    ),
    mesh=plsc.VectorSubcoreMesh(core_axis_name="c", subcore_axis_name="s",
                                num_cores=2, num_subcores=16),
    compiler_params=pltpu.CompilerParams(use_tc_tiling_on_sc=False),
    scratch_types=[
        pltpu.VMEM((3, C, 128), jnp.float32),
        pltpu.VMEM((3, C, 16), jnp.float32),
        pltpu.VMEM((3, C, 128), jnp.float32),
        pltpu.VMEM((8, C), jnp.int32),
        pltpu.VMEM((8, C), jnp.int32),
        pltpu.VMEM((8, C), jnp.int32),
        pltpu.VMEM((R * 128,), jnp.float32),
        pltpu.SemaphoreType.DMA((3,)),
        pltpu.SemaphoreType.DMA((3,)),
        pltpu.SemaphoreType.DMA((3,)),
        pltpu.SemaphoreType.DMA((3,)),
        pltpu.SemaphoreType.DMA((8,)),
        pltpu.VMEM_SHARED((NACC, 128), jnp.float32),
        pltpu.VMEM_SHARED((ND, 128), jnp.float32),
    ],
)


@jax.jit
def kernel(inputs, edge_index_rel0, edge_index_rel1, W_emb1, b_emb1, W_emb2,
           b_emb2, W_gat, a_l, a_r, W_dec1, b_dec1, W_dec2, b_dec2):
    sp = jnp.asarray(_S_PERM)
    P = jnp.eye(128, dtype=jnp.float32)[sp].T
    W2p = W_emb2[:, sp]
    b2p = b_emb2[sp].reshape(1, 128)
    Wt = W_gat[:, :, sp][:, :, :, sp]
    Wr_ = jnp.einsum('lrkhd,lrhd->lrkh', W_gat.reshape(L, R, 128, H, DH), a_r)
    Wrt = Wr_[:, :, sp, :]
    alt = a_l.transpose(0, 1, 3, 2).reshape(L, R, 128)
    Wd1p = W_dec1[sp]

    def _prep(ei):
        srcp = jnp.concatenate([ei[0], jnp.zeros((EP - E,), jnp.int32)])
        dstp = jnp.concatenate([ei[1], jnp.full((EP - E,), N, jnp.int32)])
        return srcp, dstp, dstp >> 3

    src0, dst0, d20 = _prep(edge_index_rel0)
    src1, dst1, d21 = _prep(edge_index_rel1)
    zeros = jnp.zeros((STRIPE, 128), jnp.float32)

    h1t, feat, er = _embed_dense(inputs, W_emb1, b_emb1.reshape(1, 128),
                                 W2p, b2p, P, Wt[0], Wrt[0])
    for l in range(L):
        er_pad = jnp.pad(er, ((0, 0), (0, NACC - N), (0, 0)))
        outN, outDp = _sc_edge(feat, er_pad, alt[l].reshape(R * 128),
                               src0, dst0, d20, src1, dst1, d21, zeros)
        outD = outDp.reshape(R, 2, ND * 8, 16)[:, :, :N]
        if l + 1 < L:
            h1t, feat, er = _epi_dense(outN, outD, h1t, Wt[l + 1], Wrt[l + 1])
    return _epi_decision(outN, outD, h1t, Wd1p, b_dec1.reshape(1, 128),
                         W_dec2, b_dec2.reshape(1, OUT))


# speedup vs baseline: 1.1303x; 1.0372x over previous
import functools
import jax
import jax.numpy as jnp
import numpy as np
from jax import lax
from jax.experimental import pallas as pl
from jax.experimental.pallas import tpu as pltpu
from jax.experimental.pallas import tpu_sc as plsc

N = 10000
D = 128
H = 16
DH = 8
E = 160000
L = 4
R = 2
OUT = 64

BN = 400
GRID = N // BN
C = 64
NW = 32
KSTEPS = 80
NCHUNK = KSTEPS * NW
EP = NCHUNK * C
STRIPE = 640
FCH = 80
NACC = N + 8

_p = np.arange(128)
_S_PERM = ((_p % 16) * 8 + _p // 16).tolist()


def _dense_tail(h, Wt_ref, Wrt_ref, h1t_ref, feat_ref, er_ref):
    h1t_ref[...] = h
    for r in range(R):
        feat_ref[r] = jnp.dot(h, Wt_ref[r], preferred_element_type=jnp.float32)
        er_ref[r] = jnp.dot(h, Wrt_ref[r], preferred_element_type=jnp.float32)


def _embed_dense_body(x_ref, W1_ref, b1_ref, W2p_ref, b2p_ref, P_ref,
                      Wt_ref, Wrt_ref, h1t_ref, feat_ref, er_ref):
    x = x_ref[...]
    hmid = jnp.maximum(jnp.dot(x, W1_ref[...], preferred_element_type=jnp.float32)
                       + b1_ref[...], 0.0)
    h = (jnp.dot(hmid, W2p_ref[...], preferred_element_type=jnp.float32)
         + b2p_ref[...]
         + jnp.dot(x, P_ref[...], preferred_element_type=jnp.float32))
    _dense_tail(h, Wt_ref, Wrt_ref, h1t_ref, feat_ref, er_ref)


def _epilogue(outN_ref, outD_ref, h1t_ref):
    agg = jnp.zeros((BN, 128), jnp.float32)
    for r in range(R):
        num = outN_ref[r, 0] + outN_ref[r, 1]
        den = outD_ref[r, 0] + outD_ref[r, 1]
        dent = jnp.concatenate([den] * 8, axis=1) + 1e-9
        agg = agg + num / dent
    return jnp.where(agg >= 0, agg, 0.01 * agg) + h1t_ref[...]


def _epi_dense_body(outN_ref, outD_ref, h1t_ref, Wt_ref, Wrt_ref,
                    h1t_new_ref, feat_ref, er_ref):
    h = _epilogue(outN_ref, outD_ref, h1t_ref)
    _dense_tail(h, Wt_ref, Wrt_ref, h1t_new_ref, feat_ref, er_ref)


def _epi_decision_body(outN_ref, outD_ref, h1t_ref, Wd1p_ref, bd1_ref,
                       Wd2_ref, bd2_ref, out_ref):
    h = _epilogue(outN_ref, outD_ref, h1t_ref)
    hid = jnp.maximum(jnp.dot(h, Wd1p_ref[...], preferred_element_type=jnp.float32)
                      + bd1_ref[...], 0.0)
    out_ref[...] = jnp.dot(hid, Wd2_ref[...], preferred_element_type=jnp.float32) + bd2_ref[...]


_row_spec = pl.BlockSpec((BN, 128), lambda i: (i, 0))
_row16_spec = pl.BlockSpec((BN, 16), lambda i: (i, 0))
_w_spec = pl.BlockSpec((128, 128), lambda i: (0, 0))
_b_spec = pl.BlockSpec((1, 128), lambda i: (0, 0))
_Wt_spec = pl.BlockSpec((R, 128, 128), lambda i: (0, 0, 0))
_Wrt_spec = pl.BlockSpec((R, 128, 16), lambda i: (0, 0, 0))
_feat_spec = pl.BlockSpec((R, BN, 128), lambda i: (0, i, 0))
_er_spec = pl.BlockSpec((R, BN, 16), lambda i: (0, i, 0))
_accN_spec = pl.BlockSpec((R, 2, BN, 128), lambda i: (0, 0, i, 0))
_accD_spec = pl.BlockSpec((R, 2, BN, 16), lambda i: (0, 0, i, 0))

_dense_out_shapes = (
    jax.ShapeDtypeStruct((N, 128), jnp.float32),
    jax.ShapeDtypeStruct((R, N, 128), jnp.float32),
    jax.ShapeDtypeStruct((R, N, 16), jnp.float32),
)
_dense_out_specs = (_row_spec, _feat_spec, _er_spec)

_embed_dense = pl.pallas_call(
    _embed_dense_body,
    grid=(GRID,),
    in_specs=[_row_spec, _w_spec, _b_spec, _w_spec, _b_spec, _w_spec,
              _Wt_spec, _Wrt_spec],
    out_specs=_dense_out_specs,
    out_shape=_dense_out_shapes,
)

_epi_dense = pl.pallas_call(
    _epi_dense_body,
    grid=(GRID,),
    in_specs=[_accN_spec, _accD_spec, _row_spec, _Wt_spec, _Wrt_spec],
    out_specs=_dense_out_specs,
    out_shape=_dense_out_shapes,
)

_epi_decision = pl.pallas_call(
    _epi_decision_body,
    grid=(GRID,),
    in_specs=[_accN_spec, _accD_spec, _row_spec, _w_spec, _b_spec,
              pl.BlockSpec((128, OUT), lambda i: (0, 0)),
              pl.BlockSpec((1, OUT), lambda i: (0, 0))],
    out_specs=pl.BlockSpec((BN, OUT), lambda i: (i, 0)),
    out_shape=jax.ShapeDtypeStruct((N, OUT), jnp.float32),
)


def _sc_body(feat_hbm, er_hbm, alt_hbm, src0_hbm, dst0_hbm,
             src1_hbm, dst1_hbm, zeros_hbm, zerosD_hbm,
             outN_hbm, outD_hbm,
             featbuf, erbuf, exbuf, srcb, dstb, alv,
             gsemF, gsemE, ssemN, ssemD, isem,
             accN, accD):
    idx_hbms = ((src0_hbm, dst0_hbm), (src1_hbm, dst1_hbm))
    cid = lax.axis_index("c")
    sid = lax.axis_index("s")
    wid = sid * 2 + cid

    pltpu.sync_copy(alt_hbm, alv)

    for r in range(R):
        src_hbm, dst_hbm = idx_hbms[r]

        @pl.when(sid < 15)
        def _():
            pltpu.sync_copy(zeros_hbm, accN.at[pl.ds(sid * STRIPE, STRIPE)])
            pltpu.sync_copy(zerosD_hbm, accD.at[pl.ds(sid * STRIPE, STRIPE)])

        @pl.when(sid == 15)
        def _():
            pltpu.sync_copy(zeros_hbm.at[pl.ds(0, NACC - 15 * STRIPE)],
                            accN.at[pl.ds(15 * STRIPE, NACC - 15 * STRIPE)])
            pltpu.sync_copy(zerosD_hbm.at[pl.ds(0, NACC - 15 * STRIPE)],
                            accD.at[pl.ds(15 * STRIPE, NACC - 15 * STRIPE)])
        plsc.subcore_barrier()

        def _idx_load(k):
            s = k % 8
            base = (wid + NW * k) * C
            pltpu.async_copy(src_hbm.at[pl.ds(base, C)], srcb.at[s], isem.at[s])
            pltpu.async_copy(dst_hbm.at[pl.ds(base, C)], dstb.at[s], isem.at[s])

        def _gather(k, slot):
            s = k % 8
            base = (wid + NW * k) * C
            pltpu.make_async_copy(src_hbm.at[pl.ds(base, C)], srcb.at[s],
                                  isem.at[s]).wait()
            pltpu.make_async_copy(dst_hbm.at[pl.ds(base, C)], dstb.at[s],
                                  isem.at[s]).wait()
            pltpu.async_copy(feat_hbm.at[r].at[srcb.at[s]], featbuf.at[slot],
                             gsemF.at[slot])
            pltpu.async_copy(er_hbm.at[r].at[dstb.at[s]], erbuf.at[slot],
                             gsemE.at[slot])

        def _gather_wait(k, slot):
            s = k % 8
            pltpu.make_async_copy(feat_hbm.at[r].at[srcb.at[s]], featbuf.at[slot],
                                  gsemF.at[slot]).wait()
            pltpu.make_async_copy(er_hbm.at[r].at[dstb.at[s]], erbuf.at[slot],
                                  gsemE.at[slot]).wait()

        def _scatter(k, slot):
            s = k % 8
            pltpu.async_copy(featbuf.at[slot], accN.at[dstb.at[s]], ssemN.at[slot],
                             add=True)
            pltpu.async_copy(exbuf.at[slot], accD.at[dstb.at[s]], ssemD.at[slot],
                             add=True)

        def _retire(k, slot):
            s = k % 8
            pltpu.make_async_copy(featbuf.at[slot], accN.at[dstb.at[s]],
                                  ssemN.at[slot]).wait()
            pltpu.make_async_copy(exbuf.at[slot], accD.at[dstb.at[s]],
                                  ssemD.at[slot]).wait()

        _idx_load(0)
        _idx_load(1)
        _idx_load(2)
        _gather(0, 0)

        al_vecs = [alv[pl.ds(r * 128 + j * 16, 16)] for j in range(8)]

        def _step(k, _):
            b = k % 3
            pb = (k + 1) % 3
            _gather_wait(k, b)

            def _e16(k2, _):
                for m in range(16):
                    i = k2 * 16 + m
                    fs = [featbuf[b, i, pl.ds(j * 16, 16)] for j in range(8)]
                    el = fs[0] * al_vecs[0]
                    for j in range(1, 8):
                        el = el + fs[j] * al_vecs[j]
                    e = el + erbuf[b, i]
                    e = jnp.where(e >= 0.0, e, 0.2 * e)
                    ex = jnp.exp(e)
                    for j in range(8):
                        featbuf[b, i, pl.ds(j * 16, 16)] = fs[j] * ex
                    exbuf[b, i] = ex
                return _
            lax.fori_loop(0, C // 16, _e16, None, unroll=2)

            _scatter(k, b)

            @pl.when(k >= 2)
            def _():
                _retire(k - 2, pb)

            @pl.when(k + 1 <= KSTEPS - 1)
            def _():
                _gather(k + 1, pb)

            @pl.when(k + 3 <= KSTEPS - 1)
            def _():
                _idx_load(k + 3)
            return _

        lax.fori_loop(0, KSTEPS, _step, None)

        _retire(KSTEPS - 2, (KSTEPS - 2) % 3)
        _retire(KSTEPS - 1, (KSTEPS - 1) % 3)
        plsc.subcore_barrier()

        for cpy in range(STRIPE // FCH):
            start = sid * STRIPE + cpy * FCH

            @pl.when(start < N)
            def _():
                pltpu.sync_copy(accN.at[pl.ds(start, FCH)],
                                outN_hbm.at[r, cid, pl.ds(start, FCH)])
                pltpu.sync_copy(accD.at[pl.ds(start, FCH)],
                                outD_hbm.at[r, cid, pl.ds(start, FCH)])
        plsc.subcore_barrier()


_sc_edge = pl.kernel(
    _sc_body,
    out_type=(
        jax.ShapeDtypeStruct((R, 2, N, 128), jnp.float32),
        jax.ShapeDtypeStruct((R, 2, N, 16), jnp.float32),
    ),
    mesh=plsc.VectorSubcoreMesh(core_axis_name="c", subcore_axis_name="s",
                                num_cores=2, num_subcores=16),
    compiler_params=pltpu.CompilerParams(use_tc_tiling_on_sc=False),
    scratch_types=[
        pltpu.VMEM((3, C, 128), jnp.float32),
        pltpu.VMEM((3, C, 16), jnp.float32),
        pltpu.VMEM((3, C, 16), jnp.float32),
        pltpu.VMEM((8, C), jnp.int32),
        pltpu.VMEM((8, C), jnp.int32),
        pltpu.VMEM((R * 128,), jnp.float32),
        pltpu.SemaphoreType.DMA((3,)),
        pltpu.SemaphoreType.DMA((3,)),
        pltpu.SemaphoreType.DMA((3,)),
        pltpu.SemaphoreType.DMA((3,)),
        pltpu.SemaphoreType.DMA((8,)),
        pltpu.VMEM_SHARED((NACC, 128), jnp.float32),
        pltpu.VMEM_SHARED((NACC, 16), jnp.float32),
    ],
)


@jax.jit
def kernel(inputs, edge_index_rel0, edge_index_rel1, W_emb1, b_emb1, W_emb2,
           b_emb2, W_gat, a_l, a_r, W_dec1, b_dec1, W_dec2, b_dec2):
    sp = jnp.asarray(_S_PERM)
    P = jnp.eye(128, dtype=jnp.float32)[sp].T
    W2p = W_emb2[:, sp]
    b2p = b_emb2[sp].reshape(1, 128)
    Wt = W_gat[:, :, sp][:, :, :, sp]
    Wr_ = jnp.einsum('lrkhd,lrhd->lrkh', W_gat.reshape(L, R, 128, H, DH), a_r)
    Wrt = Wr_[:, :, sp, :]
    alt = a_l.transpose(0, 1, 3, 2).reshape(L, R, 128)
    Wd1p = W_dec1[sp]

    def _prep(ei):
        srcp = jnp.concatenate([ei[0], jnp.zeros((EP - E,), jnp.int32)])
        dstp = jnp.concatenate([ei[1], jnp.full((EP - E,), N, jnp.int32)])
        return srcp, dstp

    src0, dst0 = _prep(edge_index_rel0)
    src1, dst1 = _prep(edge_index_rel1)
    zeros = jnp.zeros((STRIPE, 128), jnp.float32)
    zerosD = jnp.zeros((STRIPE, 16), jnp.float32)

    h1t, feat, er = _embed_dense(inputs, W_emb1, b_emb1.reshape(1, 128),
                                 W2p, b2p, P, Wt[0], Wrt[0])
    for l in range(L):
        er_pad = jnp.pad(er, ((0, 0), (0, NACC - N), (0, 0)))
        outN, outD = _sc_edge(feat, er_pad, alt[l].reshape(R * 128),
                              src0, dst0, src1, dst1, zeros, zerosD)
        if l + 1 < L:
            h1t, feat, er = _epi_dense(outN, outD, h1t, Wt[l + 1], Wrt[l + 1])
    return _epi_decision(outN, outD, h1t, Wd1p, b_dec1.reshape(1, 128),
                         W_dec2, b_dec2.reshape(1, OUT))


# speedup vs baseline: 1.3701x; 1.2122x over previous
import functools
import jax
import jax.numpy as jnp
import numpy as np
from jax import lax
from jax.experimental import pallas as pl
from jax.experimental.pallas import tpu as pltpu
from jax.experimental.pallas import tpu_sc as plsc

N = 10000
D = 128
H = 16
DH = 8
E = 160000
L = 4
R = 2
OUT = 64

BN = 400
GRID = N // BN
C = 80
NW = 32
KSTEPS = 63
NCHUNK = KSTEPS * NW
EP = NCHUNK * C
STRIPE = 640
FCH = 80
NACC = N + 8

_p = np.arange(128)
_S_PERM = ((_p % 16) * 8 + _p // 16).tolist()


def _dense_tail(h, Wt_ref, Wrt_ref, h1t_ref, feat_ref, er_ref):
    h1t_ref[...] = h
    for r in range(R):
        feat_ref[r] = jnp.dot(h, Wt_ref[r], preferred_element_type=jnp.float32)
        er_ref[r] = jnp.dot(h, Wrt_ref[r], preferred_element_type=jnp.float32)


def _embed_dense_body(x_ref, W1_ref, b1_ref, W2p_ref, b2p_ref, P_ref,
                      Wt_ref, Wrt_ref, h1t_ref, feat_ref, er_ref):
    x = x_ref[...]
    hmid = jnp.maximum(jnp.dot(x, W1_ref[...], preferred_element_type=jnp.float32)
                       + b1_ref[...], 0.0)
    h = (jnp.dot(hmid, W2p_ref[...], preferred_element_type=jnp.float32)
         + b2p_ref[...]
         + jnp.dot(x, P_ref[...], preferred_element_type=jnp.float32))
    _dense_tail(h, Wt_ref, Wrt_ref, h1t_ref, feat_ref, er_ref)


def _epilogue(outN_ref, outD_ref, h1t_ref):
    agg = jnp.zeros((BN, 128), jnp.float32)
    for r in range(R):
        num = outN_ref[r, 0] + outN_ref[r, 1]
        den = outD_ref[r, 0] + outD_ref[r, 1]
        dent = jnp.concatenate([den] * 8, axis=1) + 1e-9
        agg = agg + num / dent
    return jnp.where(agg >= 0, agg, 0.01 * agg) + h1t_ref[...]


def _epi_dense_body(outN_ref, outD_ref, h1t_ref, Wt_ref, Wrt_ref,
                    h1t_new_ref, feat_ref, er_ref):
    h = _epilogue(outN_ref, outD_ref, h1t_ref)
    _dense_tail(h, Wt_ref, Wrt_ref, h1t_new_ref, feat_ref, er_ref)


def _epi_decision_body(outN_ref, outD_ref, h1t_ref, Wd1p_ref, bd1_ref,
                       Wd2_ref, bd2_ref, out_ref):
    h = _epilogue(outN_ref, outD_ref, h1t_ref)
    hid = jnp.maximum(jnp.dot(h, Wd1p_ref[...], preferred_element_type=jnp.float32)
                      + bd1_ref[...], 0.0)
    out_ref[...] = jnp.dot(hid, Wd2_ref[...], preferred_element_type=jnp.float32) + bd2_ref[...]


_row_spec = pl.BlockSpec((BN, 128), lambda i: (i, 0))
_row16_spec = pl.BlockSpec((BN, 16), lambda i: (i, 0))
_w_spec = pl.BlockSpec((128, 128), lambda i: (0, 0))
_b_spec = pl.BlockSpec((1, 128), lambda i: (0, 0))
_Wt_spec = pl.BlockSpec((R, 128, 128), lambda i: (0, 0, 0))
_Wrt_spec = pl.BlockSpec((R, 128, 16), lambda i: (0, 0, 0))
_feat_spec = pl.BlockSpec((R, BN, 128), lambda i: (0, i, 0))
_er_spec = pl.BlockSpec((R, BN, 16), lambda i: (0, i, 0))
_accN_spec = pl.BlockSpec((R, 2, BN, 128), lambda i: (0, 0, i, 0))
_accD_spec = pl.BlockSpec((R, 2, BN, 16), lambda i: (0, 0, i, 0))

_dense_out_shapes = (
    jax.ShapeDtypeStruct((N, 128), jnp.float32),
    jax.ShapeDtypeStruct((R, N, 128), jnp.float32),
    jax.ShapeDtypeStruct((R, N, 16), jnp.float32),
)
_dense_out_specs = (_row_spec, _feat_spec, _er_spec)

_embed_dense = pl.pallas_call(
    _embed_dense_body,
    grid=(GRID,),
    in_specs=[_row_spec, _w_spec, _b_spec, _w_spec, _b_spec, _w_spec,
              _Wt_spec, _Wrt_spec],
    out_specs=_dense_out_specs,
    out_shape=_dense_out_shapes,
)

_epi_dense = pl.pallas_call(
    _epi_dense_body,
    grid=(GRID,),
    in_specs=[_accN_spec, _accD_spec, _row_spec, _Wt_spec, _Wrt_spec],
    out_specs=_dense_out_specs,
    out_shape=_dense_out_shapes,
)

_epi_decision = pl.pallas_call(
    _epi_decision_body,
    grid=(GRID,),
    in_specs=[_accN_spec, _accD_spec, _row_spec, _w_spec, _b_spec,
              pl.BlockSpec((128, OUT), lambda i: (0, 0)),
              pl.BlockSpec((1, OUT), lambda i: (0, 0))],
    out_specs=pl.BlockSpec((BN, OUT), lambda i: (i, 0)),
    out_shape=jax.ShapeDtypeStruct((N, OUT), jnp.float32),
)


def _sc_body(feat_hbm, er_hbm, alt_hbm, src0_hbm, dst0_hbm,
             src1_hbm, dst1_hbm, zeros_hbm, zerosD_hbm,
             outN_hbm, outD_hbm,
             featbuf, erbuf, exbuf, srcb, dstb, alv,
             gsemF, gsemE, ssemN, ssemD, isem,
             accN, accD):
    idx_hbms = ((src0_hbm, dst0_hbm), (src1_hbm, dst1_hbm))
    cid = lax.axis_index("c")
    sid = lax.axis_index("s")
    wid = sid * 2 + cid

    pltpu.sync_copy(alt_hbm, alv)

    for r in range(R):
        src_hbm, dst_hbm = idx_hbms[r]

        @pl.when(sid < 15)
        def _():
            pltpu.sync_copy(zeros_hbm, accN.at[pl.ds(sid * STRIPE, STRIPE)])
            pltpu.sync_copy(zerosD_hbm, accD.at[pl.ds(sid * STRIPE, STRIPE)])

        @pl.when(sid == 15)
        def _():
            pltpu.sync_copy(zeros_hbm.at[pl.ds(0, NACC - 15 * STRIPE)],
                            accN.at[pl.ds(15 * STRIPE, NACC - 15 * STRIPE)])
            pltpu.sync_copy(zerosD_hbm.at[pl.ds(0, NACC - 15 * STRIPE)],
                            accD.at[pl.ds(15 * STRIPE, NACC - 15 * STRIPE)])
        plsc.subcore_barrier()

        def _idx_load(k):
            s = k % 8
            base = (wid + NW * k) * C
            pltpu.async_copy(src_hbm.at[pl.ds(base, C)], srcb.at[s], isem.at[s])
            pltpu.async_copy(dst_hbm.at[pl.ds(base, C)], dstb.at[s], isem.at[s])

        def _gather(k, slot):
            s = k % 8
            base = (wid + NW * k) * C
            pltpu.make_async_copy(src_hbm.at[pl.ds(base, C)], srcb.at[s],
                                  isem.at[s]).wait()
            pltpu.make_async_copy(dst_hbm.at[pl.ds(base, C)], dstb.at[s],
                                  isem.at[s]).wait()
            pltpu.async_copy(feat_hbm.at[r].at[srcb.at[s]], featbuf.at[slot],
                             gsemF.at[slot])
            pltpu.async_copy(er_hbm.at[r].at[dstb.at[s]], erbuf.at[slot],
                             gsemE.at[slot])

        def _gather_wait(k, slot):
            s = k % 8
            pltpu.make_async_copy(feat_hbm.at[r].at[srcb.at[s]], featbuf.at[slot],
                                  gsemF.at[slot]).wait()
            pltpu.make_async_copy(er_hbm.at[r].at[dstb.at[s]], erbuf.at[slot],
                                  gsemE.at[slot]).wait()

        def _scatter(k, slot):
            s = k % 8
            pltpu.async_copy(featbuf.at[slot], accN.at[dstb.at[s]], ssemN.at[slot],
                             add=True)
            pltpu.async_copy(exbuf.at[slot], accD.at[dstb.at[s]], ssemD.at[slot],
                             add=True)

        def _retire(k, slot):
            s = k % 8
            pltpu.make_async_copy(featbuf.at[slot], accN.at[dstb.at[s]],
                                  ssemN.at[slot]).wait()
            pltpu.make_async_copy(exbuf.at[slot], accD.at[dstb.at[s]],
                                  ssemD.at[slot]).wait()

        _idx_load(0)
        _idx_load(1)
        _idx_load(2)
        _gather(0, 0)

        al_vecs = [alv[pl.ds(r * 128 + j * 16, 16)] for j in range(8)]

        def _step(k, _):
            b = k % 3
            pb = (k + 1) % 3
            _gather_wait(k, b)

            def _e16(k2, _):
                for m in range(16):
                    i = k2 * 16 + m
                    fs = [featbuf[b, i, pl.ds(j * 16, 16)] for j in range(8)]
                    el = fs[0] * al_vecs[0]
                    for j in range(1, 8):
                        el = el + fs[j] * al_vecs[j]
                    e = el + erbuf[b, i]
                    e = jnp.where(e >= 0.0, e, 0.2 * e)
                    ex = jnp.exp(e)
                    for j in range(8):
                        featbuf[b, i, pl.ds(j * 16, 16)] = fs[j] * ex
                    exbuf[b, i] = ex
                return _
            lax.fori_loop(0, C // 16, _e16, None, unroll=2)

            _scatter(k, b)

            @pl.when(k >= 2)
            def _():
                _retire(k - 2, pb)

            @pl.when(k + 1 <= KSTEPS - 1)
            def _():
                _gather(k + 1, pb)

            @pl.when(k + 3 <= KSTEPS - 1)
            def _():
                _idx_load(k + 3)
            return _

        lax.fori_loop(0, KSTEPS, _step, None)

        _retire(KSTEPS - 2, (KSTEPS - 2) % 3)
        _retire(KSTEPS - 1, (KSTEPS - 1) % 3)
        plsc.subcore_barrier()

        for cpy in range(STRIPE // FCH):
            start = sid * STRIPE + cpy * FCH

            @pl.when(start < N)
            def _():
                pltpu.sync_copy(accN.at[pl.ds(start, FCH)],
                                outN_hbm.at[r, cid, pl.ds(start, FCH)])
                pltpu.sync_copy(accD.at[pl.ds(start, FCH)],
                                outD_hbm.at[r, cid, pl.ds(start, FCH)])
        plsc.subcore_barrier()


_sc_edge = pl.kernel(
    _sc_body,
    out_type=(
        jax.ShapeDtypeStruct((R, 2, N, 128), jnp.float32),
        jax.ShapeDtypeStruct((R, 2, N, 16), jnp.float32),
    ),
    mesh=plsc.VectorSubcoreMesh(core_axis_name="c", subcore_axis_name="s",
                                num_cores=2, num_subcores=16),
    compiler_params=pltpu.CompilerParams(use_tc_tiling_on_sc=False),
    scratch_types=[
        pltpu.VMEM((3, C, 128), jnp.float32),
        pltpu.VMEM((3, C, 16), jnp.float32),
        pltpu.VMEM((3, C, 16), jnp.float32),
        pltpu.VMEM((8, C), jnp.int32),
        pltpu.VMEM((8, C), jnp.int32),
        pltpu.VMEM((R * 128,), jnp.float32),
        pltpu.SemaphoreType.DMA((3,)),
        pltpu.SemaphoreType.DMA((3,)),
        pltpu.SemaphoreType.DMA((3,)),
        pltpu.SemaphoreType.DMA((3,)),
        pltpu.SemaphoreType.DMA((8,)),
        pltpu.VMEM_SHARED((NACC, 128), jnp.float32),
        pltpu.VMEM_SHARED((NACC, 16), jnp.float32),
    ],
)


@jax.jit
def kernel(inputs, edge_index_rel0, edge_index_rel1, W_emb1, b_emb1, W_emb2,
           b_emb2, W_gat, a_l, a_r, W_dec1, b_dec1, W_dec2, b_dec2):
    sp = jnp.asarray(_S_PERM)
    P = jnp.eye(128, dtype=jnp.float32)[sp].T
    W2p = W_emb2[:, sp]
    b2p = b_emb2[sp].reshape(1, 128)
    Wt = W_gat[:, :, sp][:, :, :, sp]
    Wr_ = jnp.einsum('lrkhd,lrhd->lrkh', W_gat.reshape(L, R, 128, H, DH), a_r)
    Wrt = Wr_[:, :, sp, :]
    alt = a_l.transpose(0, 1, 3, 2).reshape(L, R, 128)
    Wd1p = W_dec1[sp]

    def _prep(ei):
        srcp = jnp.concatenate([ei[0], jnp.zeros((EP - E,), jnp.int32)])
        dstp = jnp.concatenate([ei[1], jnp.full((EP - E,), N, jnp.int32)])
        return srcp, dstp

    src0, dst0 = _prep(edge_index_rel0)
    src1, dst1 = _prep(edge_index_rel1)
    zeros = jnp.zeros((STRIPE, 128), jnp.float32)
    zerosD = jnp.zeros((STRIPE, 16), jnp.float32)

    h1t, feat, er = _embed_dense(inputs, W_emb1, b_emb1.reshape(1, 128),
                                 W2p, b2p, P, Wt[0], Wrt[0])
    for l in range(L):
        er_pad = jnp.pad(er, ((0, 0), (0, NACC - N), (0, 0)))
        outN, outD = _sc_edge(feat, er_pad, alt[l].reshape(R * 128),
                              src0, dst0, src1, dst1, zeros, zerosD)
        if l + 1 < L:
            h1t, feat, er = _epi_dense(outN, outD, h1t, Wt[l + 1], Wrt[l + 1])
    return _epi_decision(outN, outD, h1t, Wd1p, b_dec1.reshape(1, 128),
                         W_dec2, b_dec2.reshape(1, OUT))
